# baseline (speedup 1.0000x reference)
import functools

import jax
import jax.numpy as jnp
from jax import lax
from jax.experimental import pallas as pl
from jax.experimental.pallas import tpu as pltpu
from jax.experimental.pallas import tpu_sc as plsc

EMBED_DIM = 128
NBUF = 10


def _gather_t(idx_t, table):
    n_rows, n_cols = idx_t.shape
    info = plsc.get_sparse_core_info()
    num_workers = info.num_cores * info.num_subcores
    cols_per_w = n_cols // num_workers
    half = cols_per_w // 4
    n_units = n_rows * 4
    n_groups = n_units // NBUF

    mesh = plsc.VectorSubcoreMesh(core_axis_name="c", subcore_axis_name="s")

    @functools.partial(
        pl.kernel,
        mesh=mesh,
        out_type=jax.ShapeDtypeStruct((n_rows * n_cols, EMBED_DIM), jnp.float32),
        scratch_types=[
            pltpu.VMEM((n_rows, cols_per_w), jnp.int32),
        ] + [pltpu.VMEM((half, EMBED_DIM), jnp.float32)] * NBUF
          + [pltpu.SemaphoreType.DMA] * (2 * NBUF),
    )
    def k(idx_hbm, table_hbm, out_hbm, idx_v, *rest):
        bufs = rest[:NBUF]
        gsems = rest[NBUF:2 * NBUF]
        wsems = rest[2 * NBUF:]
        wid = lax.axis_index("s") * info.num_cores + lax.axis_index("c")
        col0 = wid * cols_per_w

        pltpu.sync_copy(idx_hbm.at[:, pl.ds(col0, cols_per_w)], idx_v)

        def unit_dst(u):
            return out_hbm.at[
                pl.ds((u // 4) * n_cols + col0 + (u % 4) * half, half)]

        def fire_gather(u, b):
            pltpu.async_copy(
                table_hbm.at[idx_v.at[u // 4, pl.ds((u % 4) * half, half)]],
                bufs[b], gsems[b])

        for b in range(NBUF):
            fire_gather(b, b)

        def body(g, carry):
            u0 = g * NBUF
            for b in range(NBUF):
                dst = unit_dst(u0 + b)
                pltpu.make_async_copy(dst, bufs[b], gsems[b]).wait()
                pltpu.async_copy(bufs[b], dst, wsems[b])
            for b in range(NBUF):
                pltpu.make_async_copy(bufs[b], unit_dst(u0 + b), wsems[b]).wait()

                @pl.when(u0 + b + NBUF < n_units)
                def _():
                    fire_gather(u0 + b + NBUF, b)
            return carry

        lax.fori_loop(0, n_groups, body, 0)

    return k(idx_t, table)


def kernel(x, table):
    b, s = x.shape
    idx_t = x.T.astype(jnp.int32)
    out = _gather_t(idx_t, table)
    return out.reshape(s, b, EMBED_DIM).transpose(1, 0, 2)


# speedup vs baseline: 1.0695x; 1.0695x over previous
import functools

import jax
import jax.numpy as jnp
from jax import lax
from jax.experimental import pallas as pl
from jax.experimental.pallas import tpu as pltpu
from jax.experimental.pallas import tpu_sc as plsc

EMBED_DIM = 128
NBUF = 10


def _gather_t(idx_t, table):
    n_rows, n_cols = idx_t.shape
    info = plsc.get_sparse_core_info()
    num_workers = info.num_cores * info.num_subcores
    cols_per_w = n_cols // num_workers
    half = cols_per_w // 2
    n_units = n_rows * 2
    n_groups = n_units // NBUF

    mesh = plsc.VectorSubcoreMesh(core_axis_name="c", subcore_axis_name="s")

    @functools.partial(
        pl.kernel,
        mesh=mesh,
        out_type=jax.ShapeDtypeStruct((n_rows * n_cols, EMBED_DIM), jnp.float32),
        scratch_types=[
            pltpu.VMEM((n_rows, cols_per_w), jnp.int32),
        ] + [pltpu.VMEM((half, EMBED_DIM), jnp.float32)] * NBUF
          + [pltpu.SemaphoreType.DMA] * (2 * NBUF),
    )
    def k(idx_hbm, table_hbm, out_hbm, idx_v, *rest):
        bufs = rest[:NBUF]
        gsems = rest[NBUF:2 * NBUF]
        wsems = rest[2 * NBUF:]
        wid = lax.axis_index("s") * info.num_cores + lax.axis_index("c")
        col0 = wid * cols_per_w

        pltpu.sync_copy(idx_hbm.at[:, pl.ds(col0, cols_per_w)], idx_v)

        def unit_dst(u):
            return out_hbm.at[
                pl.ds((u // 2) * n_cols + col0 + (u % 2) * half, half)]

        def fire_gather(u, b):
            pltpu.async_copy(
                table_hbm.at[idx_v.at[u // 2, pl.ds((u % 2) * half, half)]],
                bufs[b], gsems[b])

        for b in range(NBUF):
            fire_gather(b, b)

        def body(g, carry):
            u0 = g * NBUF
            for b in range(NBUF):
                dst = unit_dst(u0 + b)
                pltpu.make_async_copy(dst, bufs[b], gsems[b]).wait()
                pltpu.async_copy(bufs[b], dst, wsems[b])
            for b in range(NBUF):
                pltpu.make_async_copy(bufs[b], unit_dst(u0 + b), wsems[b]).wait()

                @pl.when(u0 + b + NBUF < n_units)
                def _():
                    fire_gather(u0 + b + NBUF, b)
            return carry

        lax.fori_loop(0, n_groups, body, 0)

    return k(idx_t, table)


def kernel(x, table):
    b, s = x.shape
    idx_t = x.T.astype(jnp.int32)
    out = _gather_t(idx_t, table)
    return out.reshape(s, b, EMBED_DIM).transpose(1, 0, 2)
